# baseline (speedup 1.0000x reference)
import functools

import jax
import jax.numpy as jnp
from jax import lax
from jax.experimental import pallas as pl
from jax.experimental.pallas import tpu as pltpu
from jax.experimental.pallas import tpu_sc as plsc

VOCAB = 1000000
HID = 64
MAXPOS = 200
B = 4096
L = 200

NC = 2
NS = 16
NW = NC * NS
BB = B // NW
NBB = BB // 16
NQ = HID // 16


def _body(ids_hbm, vec_hbm, wtab_hbm, ptab_hbm, out_hbm,
          idx_all, g0, g1, in0, in1, o0, o1, pos_v, vslab,
          gsem0, gsem1, osem0, osem1, vsem):
  wid = lax.axis_index("s") * NC + lax.axis_index("c")
  b0 = wid * BB

  gidx = (g0, g1)
  slab_in = (in0, in1)
  slab_out = (o0, o1)
  gsem = (gsem0, gsem1)
  osem = (osem0, osem1)

  iota = lax.broadcasted_iota(jnp.int32, (16,), 0)
  brow = [iota + bb * 16 for bb in range(NBB)]

  def prep_and_issue_gather(s, b):
    for bb in range(NBB):
      v = idx_all[s, pl.ds(bb * 16, 16)]
      gidx[b][pl.ds(bb * 16, 16)] = lax.shift_right_logical(v, 1)
    pltpu.async_copy(wtab_hbm.at[gidx[b]], slab_in[b], gsem[b])

  def wait_gather(b):
    pltpu.make_async_copy(wtab_hbm.at[pl.ds(0, BB)], slab_in[b],
                          gsem[b]).wait()

  def issue_out(s, b):
    pltpu.async_copy(slab_out[b], out_hbm.at[1 + s, :, pl.ds(b0, BB)],
                     osem[b])

  def wait_out(b):
    pltpu.make_async_copy(slab_out[b], out_hbm.at[0, :, pl.ds(b0, BB)],
                          osem[b]).wait()

  pltpu.sync_copy(ids_hbm.at[:, pl.ds(b0, BB)], idx_all)
  pltpu.sync_copy(ptab_hbm.at[pl.ds(0, L)], pos_v)

  pltpu.sync_copy(vec_hbm.at[:, pl.ds(b0, BB)], vslab)
  pltpu.async_copy(vslab, out_hbm.at[0, :, pl.ds(b0, BB)], vsem)

  prep_and_issue_gather(0, 0)

  @pl.loop(0, L // 2)
  def _pair(ss):
    for b in range(2):
      s = ss * 2 + b
      nb = 1 - b

      @pl.when(s + 1 < L)
      def _():
        prep_and_issue_gather(s + 1, nb)

      wait_gather(b)

      @pl.when(s >= 2)
      def _():
        wait_out(b)

      svec = jnp.full((16,), s, jnp.int32)
      half = [
          lax.shift_left(
              lax.bitwise_and(idx_all[s, pl.ds(bb * 16, 16)], 1), 6)
          for bb in range(NBB)
      ]

      @pl.loop(0, HID, unroll=2)
      def _h(h):
        hvec = jnp.full((16,), h, jnp.int32)
        p = plsc.load_gather(pos_v, [svec, hvec])
        for bb in range(NBB):
          y = plsc.load_gather(slab_in[b], [brow[bb], half[bb] + hvec]) + p
          slab_out[b][h, pl.ds(bb * 16, 16)] = y

      issue_out(s, b)

  wait_out(0)
  wait_out(1)
  pltpu.make_async_copy(vslab, out_hbm.at[0, :, pl.ds(b0, BB)], vsem).wait()


def kernel(input_ids, vectors, word_table, pos_table):
  ids_t = input_ids.T
  vec_t = vectors.T
  wtab2 = word_table.reshape(VOCAB // 2, 2 * HID)
  pos_block = lax.slice_in_dim(pos_table, 1, MAXPOS + 1, axis=0)
  mesh = plsc.VectorSubcoreMesh(core_axis_name="c", subcore_axis_name="s",
                                num_cores=NC, num_subcores=NS)
  out_t = pl.kernel(
      _body,
      out_type=jax.ShapeDtypeStruct((MAXPOS + 1, HID, B), jnp.float32),
      mesh=mesh,
      compiler_params=pltpu.CompilerParams(needs_layout_passes=False),
      scratch_types=[
          pltpu.VMEM((L, BB), jnp.int32),
          pltpu.VMEM((BB,), jnp.int32),
          pltpu.VMEM((BB,), jnp.int32),
          pltpu.VMEM((BB, 2 * HID), jnp.float32),
          pltpu.VMEM((BB, 2 * HID), jnp.float32),
          pltpu.VMEM((HID, BB), jnp.float32),
          pltpu.VMEM((HID, BB), jnp.float32),
          pltpu.VMEM((L, HID), jnp.float32),
          pltpu.VMEM((HID, BB), jnp.float32),
          pltpu.SemaphoreType.DMA,
          pltpu.SemaphoreType.DMA,
          pltpu.SemaphoreType.DMA,
          pltpu.SemaphoreType.DMA,
          pltpu.SemaphoreType.DMA,
      ],
  )(ids_t, vec_t, wtab2, pos_block)
  return jnp.transpose(out_t, (2, 0, 1))


# speedup vs baseline: 1.5033x; 1.5033x over previous
import functools

import jax
import jax.numpy as jnp
from jax import lax
from jax.experimental import pallas as pl
from jax.experimental.pallas import tpu as pltpu
from jax.experimental.pallas import tpu_sc as plsc

VOCAB = 1000000
HID = 64
MAXPOS = 200
B = 4096
L = 200

NC = 2
NS = 16
NW = NC * NS
BB = B // NW
NBB = BB // 16
NQ = HID // 16


def _body(ids_hbm, vec_hbm, wtab_hbm, ptab_hbm, out_hbm,
          idx_all, g0, g1, in0, in1, o0, o1, pos_v, vslab,
          gsem0, gsem1, osem0, osem1, vsem):
  wid = lax.axis_index("s") * NC + lax.axis_index("c")
  b0 = wid * BB

  gidx = (g0, g1)
  slab_in = (in0, in1)
  slab_out = (o0, o1)
  gsem = (gsem0, gsem1)
  osem = (osem0, osem1)

  iota = lax.broadcasted_iota(jnp.int32, (16,), 0)
  brow = [iota + bb * 16 for bb in range(NBB)]

  def prep_and_issue_gather(s, b):
    for bb in range(NBB):
      v = idx_all[s, pl.ds(bb * 16, 16)]
      gidx[b][pl.ds(bb * 16, 16)] = lax.shift_right_logical(v, 1)
    pltpu.async_copy(wtab_hbm.at[gidx[b]], slab_in[b], gsem[b])

  def wait_gather(b):
    pltpu.make_async_copy(wtab_hbm.at[pl.ds(0, BB)], slab_in[b],
                          gsem[b]).wait()

  def issue_out(s, b):
    pltpu.async_copy(slab_out[b], out_hbm.at[1 + s, :, pl.ds(b0, BB)],
                     osem[b])

  def wait_out(b):
    pltpu.make_async_copy(slab_out[b], out_hbm.at[0, :, pl.ds(b0, BB)],
                          osem[b]).wait()

  pltpu.sync_copy(ids_hbm.at[:, pl.ds(b0, BB)], idx_all)
  pltpu.sync_copy(ptab_hbm.at[pl.ds(0, L)], pos_v)

  pltpu.sync_copy(vec_hbm.at[:, pl.ds(b0, BB)], vslab)
  pltpu.async_copy(vslab, out_hbm.at[0, :, pl.ds(b0, BB)], vsem)

  prep_and_issue_gather(0, 0)

  @pl.loop(0, L // 2)
  def _pair(ss):
    for b in range(2):
      s = ss * 2 + b
      nb = 1 - b

      @pl.when(s + 1 < L)
      def _():
        prep_and_issue_gather(s + 1, nb)

      wait_gather(b)

      @pl.when(s >= 2)
      def _():
        wait_out(b)

      svec = jnp.full((16,), s, jnp.int32)
      half = [
          lax.shift_left(
              lax.bitwise_and(idx_all[s, pl.ds(bb * 16, 16)], 1), 6)
          for bb in range(NBB)
      ]

      @plsc.parallel_loop(0, HID, unroll=4)
      def _h(h):
        hvec = jnp.full((16,), h, jnp.int32)
        p = plsc.load_gather(pos_v, [svec, hvec])
        for bb in range(NBB):
          y = plsc.load_gather(slab_in[b], [brow[bb], half[bb] + hvec]) + p
          slab_out[b][h, pl.ds(bb * 16, 16)] = y

      issue_out(s, b)

  wait_out(0)
  wait_out(1)
  pltpu.make_async_copy(vslab, out_hbm.at[0, :, pl.ds(b0, BB)], vsem).wait()


def kernel(input_ids, vectors, word_table, pos_table):
  ids_t = input_ids.T
  vec_t = vectors.T
  wtab2 = word_table.reshape(VOCAB // 2, 2 * HID)
  pos_block = lax.slice_in_dim(pos_table, 1, MAXPOS + 1, axis=0)
  mesh = plsc.VectorSubcoreMesh(core_axis_name="c", subcore_axis_name="s",
                                num_cores=NC, num_subcores=NS)
  out_t = pl.kernel(
      _body,
      out_type=jax.ShapeDtypeStruct((MAXPOS + 1, HID, B), jnp.float32),
      mesh=mesh,
      compiler_params=pltpu.CompilerParams(needs_layout_passes=False),
      scratch_types=[
          pltpu.VMEM((L, BB), jnp.int32),
          pltpu.VMEM((BB,), jnp.int32),
          pltpu.VMEM((BB,), jnp.int32),
          pltpu.VMEM((BB, 2 * HID), jnp.float32),
          pltpu.VMEM((BB, 2 * HID), jnp.float32),
          pltpu.VMEM((HID, BB), jnp.float32),
          pltpu.VMEM((HID, BB), jnp.float32),
          pltpu.VMEM((L, HID), jnp.float32),
          pltpu.VMEM((HID, BB), jnp.float32),
          pltpu.SemaphoreType.DMA,
          pltpu.SemaphoreType.DMA,
          pltpu.SemaphoreType.DMA,
          pltpu.SemaphoreType.DMA,
          pltpu.SemaphoreType.DMA,
      ],
  )(ids_t, vec_t, wtab2, pos_block)
  return jnp.transpose(out_t, (2, 0, 1))
